# Initial kernel scaffold; baseline (speedup 1.0000x reference)
import jax
import jax.numpy as jnp
from jax.experimental import pallas as pl


def kernel(y_pred, y_true):
    raise NotImplementedError("write your pallas kernel here")


# speedup vs baseline: 15.8005x; 15.8005x over previous
import jax
import jax.numpy as jnp
from jax.experimental import pallas as pl

B, N, K = 64, 8192, 163


def _body(x_ref, y_ref, out_ref):
    x = x_ref[...]
    y = y_ref[...]
    loss = jnp.maximum(x, 0.0) - x * y + jnp.log1p(jnp.exp(-jnp.abs(x)))
    bits = jax.lax.bitcast_convert_type(loss, jnp.int32)

    def bit_step(i, prefix):
        cand = prefix | (1 << (30 - i))
        cnt = jnp.sum((bits >= cand).astype(jnp.int32), axis=1, keepdims=True)
        return jnp.where(cnt >= K, cand, prefix)

    prefix = jax.lax.fori_loop(0, 31, bit_step, jnp.zeros((B, 1), jnp.int32))
    thr = jax.lax.bitcast_convert_type(prefix, jnp.float32)
    gt = bits > prefix
    sum_gt = jnp.sum(jnp.where(gt, loss, 0.0), axis=1, keepdims=True)
    cnt_gt = jnp.sum(gt.astype(jnp.int32), axis=1, keepdims=True)
    row_sum = sum_gt + (K - cnt_gt).astype(jnp.float32) * thr
    out_ref[...] = (jnp.sum(row_sum) / (B * K)).reshape(1, 1)


def kernel(y_pred, y_true):
    out = pl.pallas_call(
        _body,
        out_shape=jax.ShapeDtypeStruct((1, 1), jnp.float32),
    )(y_pred, y_true)
    return out[0, 0]
